# baseline (speedup 1.0000x reference)
import jax
import jax.numpy as jnp
from jax import lax
from jax.experimental import pallas as pl
from jax.experimental.pallas import tpu as pltpu
from jax.experimental.pallas import tpu_sc as plsc

B = 16384
D = 32
L = 20
NC = 2
NS = 16
NW = NC * NS
RPW = B // NW
CHUNK = 32
NCHUNK = RPW // CHUNK
TOK_PER_CHUNK = CHUNK * L
IDX_STREAM = 128
NSTREAM = TOK_PER_CHUNK // IDX_STREAM
TITLE_STREAMS = RPW // IDX_STREAM

_MESH_KW = dict(core_axis_name="c", subcore_axis_name="s",
                num_cores=NC, num_subcores=NS)
_PARAMS = pltpu.CompilerParams(
    needs_layout_passes=False, use_tc_tiling_on_sc=False)
_PARAMS_TILED = pltpu.CompilerParams(
    needs_layout_passes=False, use_tc_tiling_on_sc=True)


def _tok_extract_body(tokT_hbm, out_hbm, ids_v, flat_v, sem):
    wid = lax.axis_index("s") * NC + lax.axis_index("c")
    base = wid * RPW
    lane = lax.iota(jnp.int32, 16)
    pltpu.sync_copy(tokT_hbm.at[:, pl.ds(base, RPW)], ids_v)
    for g in range(RPW // 16):
        dst0 = g * 16 * L
        for l in range(L):
            v = ids_v[l, pl.ds(16 * g, 16)]
            plsc.store_scatter(flat_v, [lane * L + (dst0 + l)], v)
    pltpu.sync_copy(flat_v, out_hbm.at[pl.ds(base * L, RPW * L)])


def _text_body(tok_hbm, xtab_hbm, out_hbm,
               tokidx0, xrow0, tokidx1, xrow1,
               obuf_v, tab0_v, nzf_v, rec_v, *sems):
    bufs = ((tokidx0, xrow0, sems[:NSTREAM]),
            (tokidx1, xrow1, sems[NSTREAM:]))
    wid = lax.axis_index("s") * NC + lax.axis_index("c")
    base = wid * RPW

    pltpu.sync_copy(xtab_hbm.at[pl.ds(0, 1)], tab0_v)
    t0 = (tab0_v[0, pl.ds(0, 16)], tab0_v[0, pl.ds(16, 16)])
    lane = lax.iota(jnp.int32, 16)
    one = jnp.ones((16,), jnp.float32)
    zerof = jnp.zeros((16,), jnp.float32)

    def issue(c, buf):
        tokidx_v, xrow_v, bsems = buf
        rbase = base + c * CHUNK
        pltpu.sync_copy(tok_hbm.at[pl.ds(rbase * L, TOK_PER_CHUNK)], tokidx_v)
        for j in range(NSTREAM):
            pltpu.async_copy(
                xtab_hbm.at[tokidx_v.at[pl.ds(j * IDX_STREAM, IDX_STREAM)]],
                xrow_v.at[pl.ds(j * IDX_STREAM, IDX_STREAM)], bsems[j])

    def wait(buf):
        tokidx_v, xrow_v, bsems = buf
        for j in range(NSTREAM):
            pltpu.make_async_copy(
                xtab_hbm.at[tokidx_v.at[pl.ds(j * IDX_STREAM, IDX_STREAM)]],
                xrow_v.at[pl.ds(j * IDX_STREAM, IDX_STREAM)], bsems[j]).wait()

    def compute(c, buf):
        tokidx_v, xrow_v, bsems = buf
        rbase = base + c * CHUNK
        for half in range(2):
            col0 = (lane + 16 * half) * L
            nzf = zerof
            for l in range(L):
                ids = plsc.load_gather(tokidx_v, [col0 + l])
                nzf = nzf + jnp.where(ids == 0, one, zerof)
            cntf = jnp.maximum(jnp.float32(L) - nzf, 1.0)
            nzf_v[pl.ds(16 * half + 8, 16)] = nzf
            rec_v[pl.ds(16 * half + 8, 16)] = 1.0 / cntf
        for r in range(CHUNK):
            rsplat = jnp.full((16,), r + 8, jnp.int32)
            nzf = plsc.load_gather(nzf_v, [rsplat])
            rec = plsc.load_gather(rec_v, [rsplat])
            for h in range(2):
                hs = 16 * h
                s = xrow_v[r * L, pl.ds(hs, 16)]
                for l in range(1, L):
                    s = s + xrow_v[r * L + l, pl.ds(hs, 16)]
                obuf_v[r, pl.ds(hs, 16)] = (s - nzf * t0[h]) * rec
        pltpu.sync_copy(obuf_v, out_hbm.at[pl.ds(rbase, CHUNK)])

    issue(0, bufs[0])

    def body2(i, carry):
        c0 = 2 * i
        issue(c0 + 1, bufs[1])
        wait(bufs[0])
        compute(c0, bufs[0])

        @pl.when(c0 + 2 < NCHUNK)
        def _():
            issue(c0 + 2, bufs[0])

        wait(bufs[1])
        compute(c0 + 1, bufs[1])
        return carry

    lax.fori_loop(0, NCHUNK // 2, body2, 0)


def _title_body(title_hbm, ttab_hbm, text_hbm, out_hbm,
                tidx_v, trow_v, xrow_v, obuf_v, sem, xsem):
    wid = lax.axis_index("s") * NC + lax.axis_index("c")
    base = wid * RPW
    pltpu.sync_copy(title_hbm.at[pl.ds(base, RPW)], tidx_v)
    cps = []
    for j in range(TITLE_STREAMS):
        cps.append(pltpu.async_copy(
            ttab_hbm.at[tidx_v.at[pl.ds(j * IDX_STREAM, IDX_STREAM)]],
            trow_v.at[pl.ds(j * IDX_STREAM, IDX_STREAM)], sem))
    xcp = pltpu.async_copy(text_hbm.at[pl.ds(base, RPW)], xrow_v, xsem)
    for cp in cps:
        cp.wait()
    xcp.wait()

    for r in range(RPW):
        obuf_v[r, pl.ds(0, 16)] = trow_v[r, pl.ds(0, 16)]
        obuf_v[r, pl.ds(16, 16)] = trow_v[r, pl.ds(16, 16)]
        obuf_v[r, pl.ds(32, 16)] = xrow_v[r, pl.ds(0, 16)]
        obuf_v[r, pl.ds(48, 16)] = xrow_v[r, pl.ds(16, 16)]
    pltpu.sync_copy(obuf_v, out_hbm.at[pl.ds(base, RPW)])


@jax.jit
def _movie_embed(title_ids, tokT, title_table, text_table):
    mesh = plsc.VectorSubcoreMesh(**_MESH_KW)
    tok_flat = pl.kernel(
        _tok_extract_body,
        out_type=jax.ShapeDtypeStruct((B * L,), jnp.int32),
        mesh=plsc.VectorSubcoreMesh(**_MESH_KW),
        compiler_params=_PARAMS_TILED,
        scratch_types=[
            pltpu.VMEM((L, RPW), jnp.int32),
            pltpu.VMEM((RPW * L,), jnp.int32),
            pltpu.SemaphoreType.DMA,
        ],
    )(tokT)
    text_out = pl.kernel(
        _text_body,
        out_type=jax.ShapeDtypeStruct((B, D), jnp.float32),
        mesh=mesh,
        compiler_params=_PARAMS,
        scratch_types=[
            pltpu.VMEM((TOK_PER_CHUNK,), jnp.int32),
            pltpu.VMEM((TOK_PER_CHUNK, D), jnp.float32),
            pltpu.VMEM((TOK_PER_CHUNK,), jnp.int32),
            pltpu.VMEM((TOK_PER_CHUNK, D), jnp.float32),
            pltpu.VMEM((CHUNK, D), jnp.float32),
            pltpu.VMEM((1, D), jnp.float32),
            pltpu.VMEM((CHUNK + 8,), jnp.float32),
            pltpu.VMEM((CHUNK + 8,), jnp.float32),
        ] + [pltpu.SemaphoreType.DMA] * (2 * NSTREAM),
    )(tok_flat, text_table)
    return pl.kernel(
        _title_body,
        out_type=jax.ShapeDtypeStruct((B, 2 * D), jnp.float32),
        mesh=plsc.VectorSubcoreMesh(**_MESH_KW),
        compiler_params=_PARAMS,
        scratch_types=[
            pltpu.VMEM((RPW,), jnp.int32),
            pltpu.VMEM((RPW, D), jnp.float32),
            pltpu.VMEM((RPW, D), jnp.float32),
            pltpu.VMEM((RPW, 2 * D), jnp.float32),
            pltpu.SemaphoreType.DMA,
            pltpu.SemaphoreType.DMA,
        ],
    )(title_ids, title_table, text_out)


def kernel(title_ids, token_ids, title_table, text_table):
    title_ids = title_ids.astype(jnp.int32)
    tokT = token_ids.astype(jnp.int32).T
    return _movie_embed(title_ids, tokT, title_table, text_table)


# speedup vs baseline: 1.0970x; 1.0970x over previous
import jax
import jax.numpy as jnp
from jax import lax
from jax.experimental import pallas as pl
from jax.experimental.pallas import tpu as pltpu
from jax.experimental.pallas import tpu_sc as plsc

B = 16384
D = 32
L = 20
NC = 2
NS = 16
NW = NC * NS
RPW = B // NW
CHUNK = 32
NCHUNK = RPW // CHUNK
TOK_PER_CHUNK = CHUNK * L
IDX_STREAM = 128
NSTREAM = TOK_PER_CHUNK // IDX_STREAM
TITLE_STREAMS = RPW // IDX_STREAM

_MESH_KW = dict(core_axis_name="c", subcore_axis_name="s",
                num_cores=NC, num_subcores=NS)
_PARAMS = pltpu.CompilerParams(
    needs_layout_passes=False, use_tc_tiling_on_sc=False)


def _text_body(tokT_hbm, xtab_hbm, out_hbm,
               tokidx0, flat0, xrow0, tokidx1, flat1, xrow1,
               obuf_v, tab0_v, nzf_v, rec_v, *sems):
    bufs = ((tokidx0, flat0, xrow0, sems[:NSTREAM]),
            (tokidx1, flat1, xrow1, sems[NSTREAM:]))
    wid = lax.axis_index("s") * NC + lax.axis_index("c")
    base = wid * RPW

    pltpu.sync_copy(xtab_hbm.at[pl.ds(0, 1)], tab0_v)
    t0 = (tab0_v[0, pl.ds(0, 16)], tab0_v[0, pl.ds(16, 16)])
    one = jnp.ones((16,), jnp.float32)
    zerof = jnp.zeros((16,), jnp.float32)

    def issue(c, buf):
        tokidx_v, flat_v, xrow_v, bsems = buf
        rbase = base + c * CHUNK
        pltpu.sync_copy(tokT_hbm.at[:, pl.ds(rbase, CHUNK)], tokidx_v)
        for l in range(L):
            for h in range(2):
                flat_v[pl.ds(l * CHUNK + 16 * h, 16)] = \
                    tokidx_v[l, pl.ds(16 * h, 16)]
        for j in range(NSTREAM):
            pltpu.async_copy(
                xtab_hbm.at[flat_v.at[pl.ds(j * IDX_STREAM, IDX_STREAM)]],
                xrow_v.at[pl.ds(j * IDX_STREAM, IDX_STREAM)], bsems[j])

    def wait(buf):
        tokidx_v, flat_v, xrow_v, bsems = buf
        for j in range(NSTREAM):
            pltpu.make_async_copy(
                xtab_hbm.at[flat_v.at[pl.ds(j * IDX_STREAM, IDX_STREAM)]],
                xrow_v.at[pl.ds(j * IDX_STREAM, IDX_STREAM)], bsems[j]).wait()

    def compute(c, buf):
        tokidx_v, flat_v, xrow_v, bsems = buf
        rbase = base + c * CHUNK
        for half in range(2):
            nzf = zerof
            for l in range(L):
                ids = tokidx_v[l, pl.ds(16 * half, 16)]
                nzf = nzf + jnp.where(ids == 0, one, zerof)
            cntf = jnp.maximum(jnp.float32(L) - nzf, 1.0)
            nzf_v[pl.ds(16 * half + 8, 16)] = nzf
            rec_v[pl.ds(16 * half + 8, 16)] = 1.0 / cntf
        for r in range(CHUNK):
            rsplat = jnp.full((16,), r + 8, jnp.int32)
            nzf = plsc.load_gather(nzf_v, [rsplat])
            rec = plsc.load_gather(rec_v, [rsplat])
            for h in range(2):
                hs = 16 * h
                s = xrow_v[r, pl.ds(hs, 16)]
                for l in range(1, L):
                    s = s + xrow_v[l * CHUNK + r, pl.ds(hs, 16)]
                obuf_v[r, pl.ds(hs, 16)] = (s - nzf * t0[h]) * rec
        pltpu.sync_copy(obuf_v, out_hbm.at[pl.ds(rbase, CHUNK)])

    issue(0, bufs[0])

    def body2(i, carry):
        c0 = 2 * i
        issue(c0 + 1, bufs[1])
        wait(bufs[0])
        compute(c0, bufs[0])

        @pl.when(c0 + 2 < NCHUNK)
        def _():
            issue(c0 + 2, bufs[0])

        wait(bufs[1])
        compute(c0 + 1, bufs[1])
        return carry

    lax.fori_loop(0, NCHUNK // 2, body2, 0)


def _title_body(title_hbm, ttab_hbm, text_hbm, out_hbm,
                tidx_v, trow_v, xrow_v, obuf_v, sem, xsem):
    wid = lax.axis_index("s") * NC + lax.axis_index("c")
    base = wid * RPW
    pltpu.sync_copy(title_hbm.at[pl.ds(base, RPW)], tidx_v)
    cps = []
    for j in range(TITLE_STREAMS):
        cps.append(pltpu.async_copy(
            ttab_hbm.at[tidx_v.at[pl.ds(j * IDX_STREAM, IDX_STREAM)]],
            trow_v.at[pl.ds(j * IDX_STREAM, IDX_STREAM)], sem))
    xcp = pltpu.async_copy(text_hbm.at[pl.ds(base, RPW)], xrow_v, xsem)
    for cp in cps:
        cp.wait()
    xcp.wait()

    for r in range(RPW):
        obuf_v[r, pl.ds(0, 16)] = trow_v[r, pl.ds(0, 16)]
        obuf_v[r, pl.ds(16, 16)] = trow_v[r, pl.ds(16, 16)]
        obuf_v[r, pl.ds(32, 16)] = xrow_v[r, pl.ds(0, 16)]
        obuf_v[r, pl.ds(48, 16)] = xrow_v[r, pl.ds(16, 16)]
    pltpu.sync_copy(obuf_v, out_hbm.at[pl.ds(base, RPW)])


@jax.jit
def _movie_embed(title_ids, tokT, title_table, text_table):
    mesh = plsc.VectorSubcoreMesh(**_MESH_KW)
    text_out = pl.kernel(
        _text_body,
        out_type=jax.ShapeDtypeStruct((B, D), jnp.float32),
        mesh=mesh,
        compiler_params=_PARAMS,
        scratch_types=[
            pltpu.VMEM((L, CHUNK), jnp.int32),
            pltpu.VMEM((TOK_PER_CHUNK,), jnp.int32),
            pltpu.VMEM((TOK_PER_CHUNK, D), jnp.float32),
            pltpu.VMEM((L, CHUNK), jnp.int32),
            pltpu.VMEM((TOK_PER_CHUNK,), jnp.int32),
            pltpu.VMEM((TOK_PER_CHUNK, D), jnp.float32),
            pltpu.VMEM((CHUNK, D), jnp.float32),
            pltpu.VMEM((1, D), jnp.float32),
            pltpu.VMEM((CHUNK + 8,), jnp.float32),
            pltpu.VMEM((CHUNK + 8,), jnp.float32),
        ] + [pltpu.SemaphoreType.DMA] * (2 * NSTREAM),
    )(tokT, text_table)
    return pl.kernel(
        _title_body,
        out_type=jax.ShapeDtypeStruct((B, 2 * D), jnp.float32),
        mesh=plsc.VectorSubcoreMesh(**_MESH_KW),
        compiler_params=_PARAMS,
        scratch_types=[
            pltpu.VMEM((RPW,), jnp.int32),
            pltpu.VMEM((RPW, D), jnp.float32),
            pltpu.VMEM((RPW, D), jnp.float32),
            pltpu.VMEM((RPW, 2 * D), jnp.float32),
            pltpu.SemaphoreType.DMA,
            pltpu.SemaphoreType.DMA,
        ],
    )(title_ids, title_table, text_out)


def kernel(title_ids, token_ids, title_table, text_table):
    title_ids = title_ids.astype(jnp.int32)
    tokT = token_ids.astype(jnp.int32).T
    return _movie_embed(title_ids, tokT, title_table, text_table)
